# Initial kernel scaffold; baseline (speedup 1.0000x reference)
import jax
import jax.numpy as jnp
from jax.experimental import pallas as pl


def kernel(image, audio, w1m, b1m, w2m, b2m, fc1_wt, fc1_b2, fc2_wt, fc2_b2, fc3_wt, fc3_b2, fc1s_wt, fc1s_b2, fc2s_wt, fc2s_b2, fc3s_wt, fc3s_b2, fc_wt, fc_b2):
    raise NotImplementedError("write your pallas kernel here")


# speedup vs baseline: 21.2137x; 21.2137x over previous
import numpy as np

import jax
import jax.numpy as jnp
from jax.experimental import pallas as pl
from jax.experimental.pallas import tpu as pltpu

_BT = 256


def _phase_sel(H, P, d):
    R = np.zeros((H, 5, P), np.float32)
    for i in range(P):
        for k in range(5):
            R[2 * i + d + k, k, i] = 1.0
    return jnp.asarray(R)


def _phase_weights(w4, H, P):
    Cout, Cin = w4.shape[0], w4.shape[1]
    mats = []
    for di in range(2):
        Rh = _phase_sel(H, P, di)
        for dj in range(2):
            Rw = _phase_sel(H, P, dj)
            W = jnp.einsum('ocxy,hxi,wyj->chwoij', w4, Rh, Rw)
            mats.append(W.reshape(Cin * H * H, Cout * P * P))
    return jnp.stack(mats)


def _net_kernel(x_ref, aud_ref, w1p_ref, w2p_ref, b1_ref, b2_ref,
                f1w_ref, f1b_ref, f2w_ref, f2b_ref, f3w_ref, f3b_ref,
                a1w_ref, a1b_ref, a2w_ref, a2b_ref, a3w_ref, a3b_ref,
                wfa_ref, wfi_ref, bf_ref, o_ref):
    f32 = jnp.float32

    x = x_ref[...].astype(jnp.bfloat16)
    h = jnp.dot(x, w1p_ref[0], preferred_element_type=f32)
    for p in range(1, 4):
        h = jnp.maximum(h, jnp.dot(x, w1p_ref[p], preferred_element_type=f32))
    h1 = jnp.maximum(h + b1_ref[...], 0.0).astype(jnp.bfloat16)

    h = jnp.dot(h1, w2p_ref[0], preferred_element_type=f32)
    for p in range(1, 4):
        h = jnp.maximum(h, jnp.dot(h1, w2p_ref[p], preferred_element_type=f32))
    h2 = jnp.maximum(h + b2_ref[...], 0.0)

    t = jnp.maximum(jnp.dot(h2, f1w_ref[...], preferred_element_type=f32)
                    + f1b_ref[...], 0.0)
    t = jnp.maximum(jnp.dot(t, f2w_ref[...], preferred_element_type=f32)
                    + f2b_ref[...], 0.0)
    img = jnp.dot(t, f3w_ref[...], preferred_element_type=f32) + f3b_ref[...]

    a = aud_ref[...]
    a = jnp.maximum(jnp.dot(a, a1w_ref[...], preferred_element_type=f32)
                    + a1b_ref[...], 0.0)
    a = jnp.maximum(jnp.dot(a, a2w_ref[...], preferred_element_type=f32)
                    + a2b_ref[...], 0.0)
    aud = jnp.dot(a, a3w_ref[...], preferred_element_type=f32) + a3b_ref[...]

    o_ref[...] = (jnp.dot(aud, wfa_ref[...], preferred_element_type=f32)
                  + jnp.dot(img, wfi_ref[...], preferred_element_type=f32)
                  + bf_ref[...])


def kernel(image, audio, w1m, b1m, w2m, b2m, fc1_wt, fc1_b2, fc2_wt, fc2_b2,
           fc3_wt, fc3_b2, fc1s_wt, fc1s_b2, fc2s_wt, fc2s_b2, fc3s_wt,
           fc3s_b2, fc_wt, fc_b2):
    B = image.shape[0]
    bt = _BT if B % _BT == 0 else B
    bf16 = jnp.bfloat16

    x2d = image.reshape(B, 3 * 32 * 32)
    w1p = _phase_weights(w1m.reshape(6, 3, 5, 5), 32, 14).astype(bf16)
    w2p = _phase_weights(w2m.reshape(16, 6, 5, 5), 14, 5).astype(bf16)
    b1row = jnp.repeat(b1m.reshape(6), 14 * 14).reshape(1, 1176)
    b2row = jnp.repeat(b2m.reshape(16), 5 * 5).reshape(1, 400)
    wfa, wfi = fc_wt[0:9, :], fc_wt[9:18, :]

    operands = [x2d, audio, w1p, w2p, b1row, b2row,
                fc1_wt, fc1_b2, fc2_wt, fc2_b2, fc3_wt, fc3_b2,
                fc1s_wt, fc1s_b2, fc2s_wt, fc2s_b2, fc3s_wt, fc3s_b2,
                wfa, wfi, fc_b2]
    in_specs = [pl.BlockSpec((bt, 3072), lambda b: (b, 0)),
                pl.BlockSpec((bt, 10), lambda b: (b, 0))]
    in_specs += [pl.BlockSpec(op.shape, (lambda b: (0, 0, 0)) if op.ndim == 3
                              else (lambda b: (0, 0))) for op in operands[2:]]

    return pl.pallas_call(
        _net_kernel,
        out_shape=jax.ShapeDtypeStruct((B, 9), jnp.float32),
        grid=(B // bt,),
        in_specs=in_specs,
        out_specs=pl.BlockSpec((bt, 9), lambda b: (b, 0)),
        compiler_params=pltpu.CompilerParams(
            dimension_semantics=("parallel",),
            vmem_limit_bytes=56 * 1024 * 1024,
        ),
    )(*operands)


# speedup vs baseline: 22.3860x; 1.0553x over previous
import numpy as np

import jax
import jax.numpy as jnp
from jax.experimental import pallas as pl
from jax.experimental.pallas import tpu as pltpu

_BT = 256


def _phase_sel(H, P):
    R = np.zeros((2, H, 5, P), np.float32)
    for d in range(2):
        for i in range(P):
            for k in range(5):
                R[d, 2 * i + d + k, k, i] = 1.0
    return jnp.asarray(R, jnp.bfloat16)


def _phase_weights(w4, H, P):
    Cout, Cin = w4.shape[0], w4.shape[1]
    R = _phase_sel(H, P)
    W = jnp.einsum('ocxy,dhxi,ewyj->dechwoij', w4.astype(jnp.bfloat16), R, R)
    return W.reshape(4, Cin * H * H, Cout * P * P)


def _net_kernel(x_ref, aud_ref, w1p_ref, w2p_ref, b1_ref, b2_ref,
                f1w_ref, f1b_ref, f2w_ref, f2b_ref, f3w_ref, f3b_ref,
                a1w_ref, a1b_ref, a2w_ref, a2b_ref, a3w_ref, a3b_ref,
                wfa_ref, wfi_ref, bf_ref, o_ref):
    f32 = jnp.float32

    x = x_ref[...].astype(jnp.bfloat16)
    h = jnp.dot(x, w1p_ref[0], preferred_element_type=f32)
    for p in range(1, 4):
        h = jnp.maximum(h, jnp.dot(x, w1p_ref[p], preferred_element_type=f32))
    h1 = jnp.maximum(h + b1_ref[...], 0.0).astype(jnp.bfloat16)

    h = jnp.dot(h1, w2p_ref[0], preferred_element_type=f32)
    for p in range(1, 4):
        h = jnp.maximum(h, jnp.dot(h1, w2p_ref[p], preferred_element_type=f32))
    h2 = jnp.maximum(h + b2_ref[...], 0.0)

    t = jnp.maximum(jnp.dot(h2, f1w_ref[...], preferred_element_type=f32)
                    + f1b_ref[...], 0.0)
    t = jnp.maximum(jnp.dot(t, f2w_ref[...], preferred_element_type=f32)
                    + f2b_ref[...], 0.0)
    img = jnp.dot(t, f3w_ref[...], preferred_element_type=f32) + f3b_ref[...]

    a = aud_ref[...]
    a = jnp.maximum(jnp.dot(a, a1w_ref[...], preferred_element_type=f32)
                    + a1b_ref[...], 0.0)
    a = jnp.maximum(jnp.dot(a, a2w_ref[...], preferred_element_type=f32)
                    + a2b_ref[...], 0.0)
    aud = jnp.dot(a, a3w_ref[...], preferred_element_type=f32) + a3b_ref[...]

    o_ref[...] = (jnp.dot(aud, wfa_ref[...], preferred_element_type=f32)
                  + jnp.dot(img, wfi_ref[...], preferred_element_type=f32)
                  + bf_ref[...])


def kernel(image, audio, w1m, b1m, w2m, b2m, fc1_wt, fc1_b2, fc2_wt, fc2_b2,
           fc3_wt, fc3_b2, fc1s_wt, fc1s_b2, fc2s_wt, fc2s_b2, fc3s_wt,
           fc3s_b2, fc_wt, fc_b2):
    B = image.shape[0]
    bt = _BT if B % _BT == 0 else B
    bf16 = jnp.bfloat16

    x2d = image.reshape(B, 3 * 32 * 32)
    w1p = _phase_weights(w1m.reshape(6, 3, 5, 5), 32, 14)
    w2p = _phase_weights(w2m.reshape(16, 6, 5, 5), 14, 5)
    b1row = jnp.repeat(b1m.reshape(6), 14 * 14).reshape(1, 1176)
    b2row = jnp.repeat(b2m.reshape(16), 5 * 5).reshape(1, 400)
    wfa, wfi = fc_wt[0:9, :], fc_wt[9:18, :]

    operands = [x2d, audio, w1p, w2p, b1row, b2row,
                fc1_wt, fc1_b2, fc2_wt, fc2_b2, fc3_wt, fc3_b2,
                fc1s_wt, fc1s_b2, fc2s_wt, fc2s_b2, fc3s_wt, fc3s_b2,
                wfa, wfi, fc_b2]
    in_specs = [pl.BlockSpec((bt, 3072), lambda b: (b, 0)),
                pl.BlockSpec((bt, 10), lambda b: (b, 0))]
    in_specs += [pl.BlockSpec(op.shape, (lambda b: (0, 0, 0)) if op.ndim == 3
                              else (lambda b: (0, 0))) for op in operands[2:]]

    return pl.pallas_call(
        _net_kernel,
        out_shape=jax.ShapeDtypeStruct((B, 9), jnp.float32),
        grid=(B // bt,),
        in_specs=in_specs,
        out_specs=pl.BlockSpec((bt, 9), lambda b: (b, 0)),
        compiler_params=pltpu.CompilerParams(
            dimension_semantics=("parallel",),
            vmem_limit_bytes=56 * 1024 * 1024,
        ),
    )(*operands)
